# Initial kernel scaffold; baseline (speedup 1.0000x reference)
import jax
import jax.numpy as jnp
from jax.experimental import pallas as pl


def kernel(x, table):
    raise NotImplementedError("write your pallas kernel here")


# speedup vs baseline: 7.5618x; 7.5618x over previous
import functools

import jax
import jax.numpy as jnp
from jax import lax
from jax.experimental import pallas as pl
from jax.experimental.pallas import tpu as pltpu
from jax.experimental.pallas import tpu_sc as plsc

D = 128
SCALE = float(128.0 ** 0.5)

NC = 2
NS = 16
NW = NC * NS

K = 8
CHUNK = K * 128
HALF = CHUNK // 2


def _scale_body(t_ref, o_ref):
    o_ref[...] = t_ref[...] * SCALE


def _scale_table(table):
    v, d = table.shape
    blk = 4000
    return pl.pallas_call(
        _scale_body,
        grid=(v // blk,),
        in_specs=[pl.BlockSpec((blk, d), lambda i: (i, 0))],
        out_specs=pl.BlockSpec((blk, d), lambda i: (i, 0)),
        out_shape=jax.ShapeDtypeStruct((v, d), jnp.float32),
    )(table)


def _make_gather(B):
    assert B % (NW * CHUNK) == 0
    blocks = B // (NW * CHUNK)
    idx_rows_per_w = blocks * K
    rows_per_w = blocks * CHUNK

    mesh = plsc.VectorSubcoreMesh(core_axis_name="c", subcore_axis_name="s")

    @functools.partial(
        pl.kernel,
        mesh=mesh,
        out_type=jax.ShapeDtypeStruct((B, D), jnp.float32),
        scratch_types=[
            pltpu.VMEM((K, 128), jnp.int32),
            pltpu.VMEM((HALF, D), jnp.float32),
            pltpu.SemaphoreType.DMA,
        ],
    )
    def gather(tab_hbm, idx_hbm, out_hbm, idx_v, rows_v, sem):
        wid = lax.axis_index("s") * NC + lax.axis_index("c")

        def blk(b, carry):
            ib = wid * idx_rows_per_w + b * K
            pltpu.sync_copy(idx_hbm.at[pl.ds(ib, K)], idx_v)
            for h in range(2):
                copies = [
                    pltpu.async_copy(
                        tab_hbm.at[idx_v.at[h * (K // 2) + j]],
                        rows_v.at[pl.ds(j * 128, 128)],
                        sem,
                    )
                    for j in range(K // 2)
                ]
                for c in copies:
                    c.wait()
                ob = wid * rows_per_w + b * CHUNK + h * HALF
                pltpu.sync_copy(rows_v, out_hbm.at[pl.ds(ob, HALF)])
            return carry

        lax.fori_loop(0, blocks, blk, 0)

    return gather


def kernel(x, table):
    bsz, seq = x.shape
    B = bsz * seq
    scaled = _scale_table(table)
    idx = x.reshape(B // 128, 128).astype(jnp.int32)
    out = _make_gather(B)(scaled, idx)
    return out.reshape(bsz, seq, D)


# speedup vs baseline: 8.0703x; 1.0673x over previous
import functools

import jax
import jax.numpy as jnp
from jax import lax
from jax.experimental import pallas as pl
from jax.experimental.pallas import tpu as pltpu
from jax.experimental.pallas import tpu_sc as plsc

D = 128
SCALE = float(128.0 ** 0.5)

NC = 2
NS = 16
NW = NC * NS

K = 8
CHUNK = K * 128
STEP = 256
NSTEP = CHUNK // STEP


def _scale_body(t_ref, o_ref):
    o_ref[...] = t_ref[...] * SCALE


def _scale_table(table):
    v, d = table.shape
    blk = 4000
    return pl.pallas_call(
        _scale_body,
        grid=(v // blk,),
        in_specs=[pl.BlockSpec((blk, d), lambda i: (i, 0))],
        out_specs=pl.BlockSpec((blk, d), lambda i: (i, 0)),
        out_shape=jax.ShapeDtypeStruct((v, d), jnp.float32),
    )(table)


def _make_gather(B):
    assert B % (NW * CHUNK) == 0
    blocks = B // (NW * CHUNK)
    idx_rows_per_w = blocks * K
    rows_per_w = blocks * CHUNK

    mesh = plsc.VectorSubcoreMesh(core_axis_name="c", subcore_axis_name="s")

    @functools.partial(
        pl.kernel,
        mesh=mesh,
        out_type=jax.ShapeDtypeStruct((B, D), jnp.float32),
        scratch_types=[
            pltpu.VMEM((K, 128), jnp.int32),
            pltpu.VMEM((2, STEP, D), jnp.float32),
            pltpu.SemaphoreType.DMA,
            pltpu.SemaphoreType.DMA,
            pltpu.SemaphoreType.DMA,
        ],
    )
    def gather(tab_hbm, idx_hbm, out_hbm, idx_v, rows_v, sem_in, so0, so1):
        wid = lax.axis_index("s") * NC + lax.axis_index("c")
        sem_out = (so0, so1)

        def drain_out(p):
            pltpu.make_async_copy(
                out_hbm.at[pl.ds(0, STEP)], rows_v.at[p], sem_out[p]
            ).wait()

        def blk(b, carry):
            ib = wid * idx_rows_per_w + b * K
            pltpu.sync_copy(idx_hbm.at[pl.ds(ib, K)], idx_v)
            for h in range(NSTEP):
                p = h % 2
                if h >= 2:
                    drain_out(p)
                else:
                    @pl.when(b > 0)
                    def _():
                        drain_out(p)
                copies = [
                    pltpu.async_copy(
                        tab_hbm.at[idx_v.at[h * 2 + j]],
                        rows_v.at[p, pl.ds(j * 128, 128)],
                        sem_in,
                    )
                    for j in range(2)
                ]
                for c in copies:
                    c.wait()
                ob = wid * rows_per_w + b * CHUNK + h * STEP
                pltpu.async_copy(
                    rows_v.at[p], out_hbm.at[pl.ds(ob, STEP)], sem_out[p]
                )
            return carry

        lax.fori_loop(0, blocks, blk, 0)
        drain_out(0)
        drain_out(1)

    return gather


def kernel(x, table):
    bsz, seq = x.shape
    B = bsz * seq
    scaled = _scale_table(table)
    idx = x.reshape(B // 128, 128).astype(jnp.int32)
    out = _make_gather(B)(scaled, idx)
    return out.reshape(bsz, seq, D)


# speedup vs baseline: 8.2739x; 1.0252x over previous
import functools

import jax
import jax.numpy as jnp
from jax import lax
from jax.experimental import pallas as pl
from jax.experimental.pallas import tpu as pltpu
from jax.experimental.pallas import tpu_sc as plsc

D = 128
SCALE = float(128.0 ** 0.5)

NC = 2
NS = 16
NW = NC * NS

K = 40
CHUNK = K * 128
STEP = 256
NSTEP = CHUNK // STEP


def _scale_body(t_ref, o_ref):
    o_ref[...] = t_ref[...] * SCALE


def _scale_table(table):
    v, d = table.shape
    blk = 4000
    return pl.pallas_call(
        _scale_body,
        grid=(v // blk,),
        in_specs=[pl.BlockSpec((blk, d), lambda i: (i, 0))],
        out_specs=pl.BlockSpec((blk, d), lambda i: (i, 0)),
        out_shape=jax.ShapeDtypeStruct((v, d), jnp.float32),
    )(table)


def _make_gather(B):
    assert B % (NW * CHUNK) == 0
    blocks = B // (NW * CHUNK)
    idx_rows_per_w = blocks * K
    rows_per_w = blocks * CHUNK

    mesh = plsc.VectorSubcoreMesh(core_axis_name="c", subcore_axis_name="s")

    @functools.partial(
        pl.kernel,
        mesh=mesh,
        out_type=jax.ShapeDtypeStruct((B, D), jnp.float32),
        scratch_types=[
            pltpu.VMEM((K, 128), jnp.int32),
            pltpu.VMEM((2, STEP, D), jnp.float32),
            pltpu.SemaphoreType.DMA,
            pltpu.SemaphoreType.DMA,
            pltpu.SemaphoreType.DMA,
            pltpu.SemaphoreType.DMA,
        ],
    )
    def gather(tab_hbm, idx_hbm, out_hbm, idx_v, rows_v, si0, si1, so0, so1):
        wid = lax.axis_index("s") * NC + lax.axis_index("c")
        sem_in = (si0, si1)
        sem_out = (so0, so1)

        def drain_out(p):
            pltpu.make_async_copy(
                out_hbm.at[pl.ds(0, STEP)], rows_v.at[p], sem_out[p]
            ).wait()

        def wait_in(p):
            pltpu.make_async_copy(
                tab_hbm.at[pl.ds(0, STEP)], rows_v.at[p], sem_in[p]
            ).wait()

        def blk(b, carry):
            ib = wid * idx_rows_per_w + b * K
            pltpu.sync_copy(idx_hbm.at[pl.ds(ib, K)], idx_v)
            for h in range(NSTEP):
                p = h % 2
                if h >= 2:
                    drain_out(p)
                else:
                    @pl.when(b > 0)
                    def _():
                        drain_out(p)
                for j in range(2):
                    pltpu.async_copy(
                        tab_hbm.at[idx_v.at[h * 2 + j]],
                        rows_v.at[p, pl.ds(j * 128, 128)],
                        sem_in[p],
                    )
                ob_prev = wid * rows_per_w + b * CHUNK + (h - 1) * STEP

                def retire(ob=ob_prev, q=1 - p):
                    wait_in(q)
                    pltpu.async_copy(
                        rows_v.at[q], out_hbm.at[pl.ds(ob, STEP)], sem_out[q]
                    )

                if h >= 1:
                    retire()
                else:
                    @pl.when(b > 0)
                    def _():
                        retire()
            return carry

        lax.fori_loop(0, blocks, blk, 0)
        wait_in(1)
        pltpu.async_copy(
            rows_v.at[1],
            out_hbm.at[pl.ds((wid + 1) * rows_per_w - STEP, STEP)],
            sem_out[1],
        )
        drain_out(0)
        drain_out(1)

    return gather


def kernel(x, table):
    bsz, seq = x.shape
    B = bsz * seq
    scaled = _scale_table(table)
    idx = x.reshape(B // 128, 128).astype(jnp.int32)
    out = _make_gather(B)(scaled, idx)
    return out.reshape(bsz, seq, D)


# speedup vs baseline: 8.2925x; 1.0023x over previous
import functools

import jax
import jax.numpy as jnp
from jax import lax
from jax.experimental import pallas as pl
from jax.experimental.pallas import tpu as pltpu
from jax.experimental.pallas import tpu_sc as plsc

D = 128
SCALE = float(128.0 ** 0.5)

NC = 2
NS = 16
NW = NC * NS

STEP = 128
NBUF = 5
W = 3


def _scale_body(t_ref, o_ref):
    o_ref[...] = t_ref[...] * SCALE


def _scale_table(table):
    v, d = table.shape
    blk = 4000
    return pl.pallas_call(
        _scale_body,
        grid=(v // blk,),
        in_specs=[pl.BlockSpec((blk, d), lambda i: (i, 0))],
        out_specs=pl.BlockSpec((blk, d), lambda i: (i, 0)),
        out_shape=jax.ShapeDtypeStruct((v, d), jnp.float32),
    )(table)


def _make_gather(B):
    assert B % (NW * STEP * NBUF) == 0
    steps = B // (NW * STEP)
    idx_rows_per_w = steps
    rows_per_w = steps * STEP
    outer = steps // NBUF

    mesh = plsc.VectorSubcoreMesh(core_axis_name="c", subcore_axis_name="s")

    @functools.partial(
        pl.kernel,
        mesh=mesh,
        out_type=jax.ShapeDtypeStruct((B, D), jnp.float32),
        scratch_types=[
            pltpu.VMEM((idx_rows_per_w, 128), jnp.int32),
            pltpu.VMEM((NBUF, STEP, D), jnp.float32),
        ] + [pltpu.SemaphoreType.DMA] * (2 * NBUF),
    )
    def gather(tab_hbm, idx_hbm, out_hbm, idx_v, rows_v, *sems):
        sem_in = sems[:NBUF]
        sem_out = sems[NBUF:]
        wid = lax.axis_index("s") * NC + lax.axis_index("c")
        obase = wid * rows_per_w

        pltpu.sync_copy(idx_hbm.at[pl.ds(wid * idx_rows_per_w, idx_rows_per_w)], idx_v)

        def drain_out(q):
            pltpu.make_async_copy(
                out_hbm.at[pl.ds(0, STEP)], rows_v.at[q], sem_out[q]
            ).wait()

        def fire(s, q):
            pltpu.async_copy(tab_hbm.at[idx_v.at[s]], rows_v.at[q], sem_in[q])

        def retire(s, q):
            pltpu.make_async_copy(
                tab_hbm.at[pl.ds(0, STEP)], rows_v.at[q], sem_in[q]
            ).wait()
            pltpu.async_copy(
                rows_v.at[q], out_hbm.at[pl.ds(obase + s * STEP, STEP)], sem_out[q]
            )

        def body(it, carry):
            for h in range(NBUF):
                s = it * NBUF + h
                @pl.when(it > 0)
                def _(h=h):
                    drain_out(h)
                fire(s, h)
                if h >= W:
                    retire(s - W, (h - W) % NBUF)
                else:
                    @pl.when(it > 0)
                    def _(s=s, h=h):
                        retire(s - W, (h - W) % NBUF)
            return carry

        lax.fori_loop(0, outer, body, 0)
        for w in range(W, 0, -1):
            retire(steps - w, (steps - w) % NBUF)
        for q in range(NBUF):
            drain_out(q)

    return gather


def kernel(x, table):
    bsz, seq = x.shape
    B = bsz * seq
    scaled = _scale_table(table)
    idx = x.reshape(B // 128, 128).astype(jnp.int32)
    out = _make_gather(B)(scaled, idx)
    return out.reshape(bsz, seq, D)
